# Initial kernel scaffold; baseline (speedup 1.0000x reference)
import functools

import jax
import jax.numpy as jnp
from jax import lax
from jax.experimental import pallas as pl
from jax.experimental.pallas import tpu as pltpu
from jax.experimental.pallas import tpu_sc as plsc

N = 10000
E = 320000
D = 128
LANES = 16
NC = 2
NS = 16
NW = NC * NS
EPT = E // NW
CHUNK = 40
NCHUNK = EPT // CHUNK
NPAIR = NCHUNK // 2
RPT = N // NS


def _sc_aggregate(x, col2d, row2d, w2d):
    mesh = plsc.VectorSubcoreMesh(core_axis_name="c", subcore_axis_name="s")

    @functools.partial(
        pl.kernel,
        mesh=mesh,
        out_type=jax.ShapeDtypeStruct((NC * N, D), jnp.float32),
        scratch_types=[
            pltpu.VMEM_SHARED((N, D), jnp.float32),
            pltpu.VMEM((NCHUNK, CHUNK), jnp.int32),
            pltpu.VMEM((NCHUNK, CHUNK), jnp.int32),
            pltpu.VMEM((NCHUNK, CHUNK), jnp.float32),
            pltpu.VMEM((CHUNK, D), jnp.float32),
            pltpu.VMEM((CHUNK, D), jnp.float32),
            pltpu.SemaphoreType.DMA,
            pltpu.SemaphoreType.DMA,
        ],
    )
    def k(x_hbm, col_hbm, row_hbm, w_hbm, out_hbm,
          acc, col_v, row_v, w_v, buf0, buf1, sem0, sem1):
        cid = lax.axis_index("c")
        sid = lax.axis_index("s")
        wid = cid * NS + sid
        cbase = wid * NCHUNK

        pltpu.sync_copy(col_hbm.at[pl.ds(cbase, NCHUNK)], col_v)
        pltpu.sync_copy(row_hbm.at[pl.ds(cbase, NCHUNK)], row_v)
        pltpu.sync_copy(w_hbm.at[pl.ds(cbase, NCHUNK)], w_v)

        zeros = jnp.zeros((LANES,), jnp.float32)
        for j in range(CHUNK):
            for c in range(D // LANES):
                buf0[j, pl.ds(c * LANES, LANES)] = zeros
        r0 = sid * RPT
        for i in range(RPT // CHUNK):
            pltpu.sync_copy(buf0, acc.at[pl.ds(r0 + i * CHUNK, CHUNK)])
        rem = RPT % CHUNK
        if rem:
            pltpu.sync_copy(buf0.at[pl.ds(0, rem)],
                            acc.at[pl.ds(r0 + (RPT // CHUNK) * CHUNK, rem)])
        plsc.subcore_barrier()

        def gather_start(g, buf, sem):
            pltpu.make_async_copy(x_hbm.at[col_v.at[g]], buf, sem).start()

        def gather_wait(g, buf, sem):
            pltpu.make_async_copy(x_hbm.at[col_v.at[g]], buf, sem).wait()

        def do_chunk(g, buf):
            g16 = jnp.broadcast_to(g, (LANES,)).astype(jnp.int32)
            for j in range(CHUNK):
                j16 = jnp.full((LANES,), j, jnp.int32)
                wsplat = plsc.load_gather(w_v, [g16, j16])
                for c in range(D // LANES):
                    sl = pl.ds(c * LANES, LANES)
                    buf[j, sl] = buf[j, sl] * wsplat
            pltpu.sync_copy(buf, acc.at[row_v.at[g]], add=True)

        gather_start(0, buf0, sem0)

        def body(p, carry):
            g0 = p * 2
            gather_start(g0 + 1, buf1, sem1)
            gather_wait(g0, buf0, sem0)
            do_chunk(g0, buf0)

            @pl.when(p < NPAIR - 1)
            def _():
                gather_start(g0 + 2, buf0, sem0)

            gather_wait(g0 + 1, buf1, sem1)
            do_chunk(g0 + 1, buf1)
            return carry

        lax.fori_loop(0, NPAIR, body, 0)

        plsc.subcore_barrier()
        pltpu.sync_copy(acc.at[pl.ds(r0, RPT)],
                        out_hbm.at[pl.ds(cid * N + r0, RPT)])

    return k(x, col2d, row2d, w2d)


def _tc_body(x_ref, p0_ref, p1_ref, ws_ref, wn_ref, b_ref, o_ref):
    dn = (((1,), (1,)), ((), ()))
    o_ref[...] = (
        lax.dot_general(x_ref[...], ws_ref[...], dn,
                        preferred_element_type=jnp.float32)
        + b_ref[...]
        + lax.dot_general(p0_ref[...] + p1_ref[...], wn_ref[...], dn,
                          preferred_element_type=jnp.float32)
    )


def _tc_combine(x, partial, W_self, W_neigh, b2d):
    BM = 1000
    nblk = N // BM
    return pl.pallas_call(
        _tc_body,
        grid=(nblk,),
        in_specs=[
            pl.BlockSpec((BM, D), lambda i: (i, 0)),
            pl.BlockSpec((BM, D), lambda i: (i, 0)),
            pl.BlockSpec((BM, D), lambda i, _n=nblk: (i + _n, 0)),
            pl.BlockSpec((D, D), lambda i: (0, 0)),
            pl.BlockSpec((D, D), lambda i: (0, 0)),
            pl.BlockSpec((1, D), lambda i: (0, 0)),
        ],
        out_specs=pl.BlockSpec((BM, D), lambda i: (i, 0)),
        out_shape=jax.ShapeDtypeStruct((N, D), jnp.float32),
    )(x, partial, partial, W_self, W_neigh, b2d)


def kernel(x, edge_index, edge_weight, W_self, b_self, W_neigh):
    row2d = edge_index[0].astype(jnp.int32).reshape(NW * NCHUNK, CHUNK)
    col2d = edge_index[1].astype(jnp.int32).reshape(NW * NCHUNK, CHUNK)
    w2d = edge_weight.astype(jnp.float32).reshape(NW * NCHUNK, CHUNK)
    partial = _sc_aggregate(x, col2d, row2d, w2d)
    return _tc_combine(x, partial, W_self, W_neigh, b_self.reshape(1, D))


# speedup vs baseline: 10.0710x; 10.0710x over previous
import functools

import jax
import jax.numpy as jnp
from jax import lax
from jax.experimental import pallas as pl
from jax.experimental.pallas import tpu as pltpu
from jax.experimental.pallas import tpu_sc as plsc

N = 10000
E = 320000
D = 128
LANES = 16
NC = 2
NS = 16
NW = NC * NS
EPT = E // NW
CHUNK = 80
NCHUNK = EPT // CHUNK
NPAIR = (NCHUNK - 1) // 2
RPT = 632
RPT_LAST = N - (NS - 1) * RPT


def _sc_aggregate(x, col1d, row1d, w1d):
    mesh = plsc.VectorSubcoreMesh(core_axis_name="c", subcore_axis_name="s")

    @functools.partial(
        pl.kernel,
        mesh=mesh,
        out_type=jax.ShapeDtypeStruct((NC * N, D), jnp.float32),
        scratch_types=[
            pltpu.VMEM_SHARED((N, D), jnp.float32),
            pltpu.VMEM((EPT,), jnp.int32),
            pltpu.VMEM((EPT,), jnp.int32),
            pltpu.VMEM((EPT,), jnp.float32),
            pltpu.VMEM((CHUNK, D), jnp.float32),
            pltpu.VMEM((CHUNK, D), jnp.float32),
            pltpu.VMEM((CHUNK,), jnp.int32),
            pltpu.VMEM((CHUNK,), jnp.int32),
            pltpu.SemaphoreType.DMA,
            pltpu.SemaphoreType.DMA,
            pltpu.SemaphoreType.DMA,
            pltpu.SemaphoreType.DMA,
        ],
    )
    def k(x_hbm, col_hbm, row_hbm, w_hbm, out_hbm,
          acc, col_v, row_v, w_v, buf0, buf1, ridx0, ridx1,
          gsem0, gsem1, ssem0, ssem1):
        cid = lax.axis_index("c")
        sid = lax.axis_index("s")
        wid = cid * NS + sid

        pltpu.sync_copy(col_hbm.at[pl.ds(wid * EPT, EPT)], col_v)
        pltpu.sync_copy(row_hbm.at[pl.ds(wid * EPT, EPT)], row_v)
        pltpu.sync_copy(w_hbm.at[pl.ds(wid * EPT, EPT)], w_v)

        zeros = jnp.zeros((LANES,), jnp.float32)

        def zbody(j, c_):
            for c in range(D // LANES):
                buf0[j, pl.ds(c * LANES, LANES)] = zeros
            return c_

        lax.fori_loop(0, CHUNK, zbody, 0)
        r0 = sid * RPT

        def zero_rows(base, nrows):
            for i in range(nrows // CHUNK):
                pltpu.sync_copy(buf0, acc.at[pl.ds(base + i * CHUNK, CHUNK)])
            rem = nrows % CHUNK
            if rem:
                pltpu.sync_copy(buf0.at[pl.ds(0, rem)],
                                acc.at[pl.ds(base + (nrows // CHUNK) * CHUNK,
                                             rem)])

        zero_rows(r0, RPT_LAST)

        @pl.when(sid < NS - 1)
        def _():
            zero_rows(r0 + RPT_LAST, RPT - RPT_LAST)

        plsc.subcore_barrier()

        def gather_start(g, buf, sem):
            pltpu.make_async_copy(
                x_hbm.at[col_v.at[pl.ds(g * CHUNK, CHUNK)]], buf, sem).start()

        def gather_wait(g, buf, sem):
            pltpu.make_async_copy(
                x_hbm.at[col_v.at[pl.ds(g * CHUNK, CHUNK)]], buf, sem).wait()

        def do_chunk(g, buf, ridx, ssem):
            wbase = g * CHUNK
            for jj in range(CHUNK // LANES):
                w16 = w_v[pl.ds(wbase + jj * LANES, LANES)]
                for l in range(LANES):
                    j = jj * LANES + l
                    wsplat = jnp.broadcast_to(w16[l], (LANES,))
                    for c in range(D // LANES):
                        sl = pl.ds(c * LANES, LANES)
                        buf[j, sl] = buf[j, sl] * wsplat
            for o in range(0, CHUNK, LANES):
                ridx[pl.ds(o, LANES)] = row_v[pl.ds(wbase + o, LANES)]
            pltpu.async_copy(buf, acc.at[ridx], ssem, add=True)

        def scatter_wait(buf, ridx, ssem):
            pltpu.make_async_copy(buf, acc.at[ridx], ssem).wait()

        gather_start(0, buf0, gsem0)

        def body(p, carry):
            g0 = p * 2

            @pl.when(p > 0)
            def _():
                scatter_wait(buf1, ridx1, ssem1)

            gather_start(g0 + 1, buf1, gsem1)
            gather_wait(g0, buf0, gsem0)
            do_chunk(g0, buf0, ridx0, ssem0)
            gather_wait(g0 + 1, buf1, gsem1)
            do_chunk(g0 + 1, buf1, ridx1, ssem1)
            scatter_wait(buf0, ridx0, ssem0)
            gather_start(g0 + 2, buf0, gsem0)
            return carry

        lax.fori_loop(0, NPAIR, body, 0)

        gather_wait(NCHUNK - 1, buf0, gsem0)
        do_chunk(NCHUNK - 1, buf0, ridx0, ssem0)
        scatter_wait(buf0, ridx0, ssem0)
        scatter_wait(buf1, ridx1, ssem1)

        plsc.subcore_barrier()

        @pl.when(sid < NS - 1)
        def _():
            pltpu.sync_copy(acc.at[pl.ds(r0, RPT)],
                            out_hbm.at[pl.ds(cid * N + r0, RPT)])

        @pl.when(sid == NS - 1)
        def _():
            pltpu.sync_copy(acc.at[pl.ds(r0, RPT_LAST)],
                            out_hbm.at[pl.ds(cid * N + r0, RPT_LAST)])

    return k(x, col1d, row1d, w1d)


def _tc_body(x_ref, p0_ref, p1_ref, ws_ref, wn_ref, b_ref, o_ref):
    dn = (((1,), (1,)), ((), ()))
    o_ref[...] = (
        lax.dot_general(x_ref[...], ws_ref[...], dn,
                        preferred_element_type=jnp.float32)
        + b_ref[...]
        + lax.dot_general(p0_ref[...] + p1_ref[...], wn_ref[...], dn,
                          preferred_element_type=jnp.float32)
    )


def _tc_combine(x, partial, W_self, W_neigh, b2d):
    BM = 1000
    nblk = N // BM
    return pl.pallas_call(
        _tc_body,
        grid=(nblk,),
        in_specs=[
            pl.BlockSpec((BM, D), lambda i: (i, 0)),
            pl.BlockSpec((BM, D), lambda i: (i, 0)),
            pl.BlockSpec((BM, D), lambda i, _n=nblk: (i + _n, 0)),
            pl.BlockSpec((D, D), lambda i: (0, 0)),
            pl.BlockSpec((D, D), lambda i: (0, 0)),
            pl.BlockSpec((1, D), lambda i: (0, 0)),
        ],
        out_specs=pl.BlockSpec((BM, D), lambda i: (i, 0)),
        out_shape=jax.ShapeDtypeStruct((N, D), jnp.float32),
    )(x, partial, partial, W_self, W_neigh, b2d)


def kernel(x, edge_index, edge_weight, W_self, b_self, W_neigh):
    row1d = edge_index[0].astype(jnp.int32)
    col1d = edge_index[1].astype(jnp.int32)
    w1d = edge_weight.astype(jnp.float32)
    partial = _sc_aggregate(x, col1d, row1d, w1d)
    return _tc_combine(x, partial, W_self, W_neigh, b_self.reshape(1, D))
